# Initial kernel scaffold; baseline (speedup 1.0000x reference)
import functools

import jax
import jax.numpy as jnp
from jax import lax
from jax.experimental import pallas as pl
from jax.experimental.pallas import tpu as pltpu
from jax.experimental.pallas import tpu_sc as plsc

_NC = 2
_NS = 16
_L = 16

_BLOCK_ROWS = 1024
_IDX_ROW = 128


def _sq_sum_body(x_ref, out_ref):
    @pl.when(pl.program_id(0) == 0)
    def _init():
        out_ref[0, 0] = jnp.float32(0.0)

    x = x_ref[...]
    out_ref[0, 0] += jnp.sum(x * x)


@functools.partial(jax.jit, static_argnums=(1,))
def _sq_sum(x, block_rows):
    b, t = x.shape
    return pl.pallas_call(
        _sq_sum_body,
        grid=(b // block_rows,),
        in_specs=[pl.BlockSpec((block_rows, t), lambda i: (i, 0))],
        out_specs=pl.BlockSpec(memory_space=pltpu.SMEM),
        out_shape=jax.ShapeDtypeStruct((1, 1), jnp.float32),
    )(x)


@functools.cache
def _make_gather_sum(b, t):
    nw = _NC * _NS
    bw = b // nw
    nchunk = bw // _IDX_ROW
    mesh = plsc.VectorSubcoreMesh(core_axis_name="c", subcore_axis_name="s")

    @functools.partial(
        pl.kernel,
        mesh=mesh,
        out_type=jax.ShapeDtypeStruct((nw, _L), jnp.float32),
        scratch_types=[
            pltpu.VMEM((bw,), jnp.int32),
            pltpu.VMEM((nchunk, _IDX_ROW), jnp.int32),
            pltpu.VMEM((nchunk, _IDX_ROW), jnp.float32),
            pltpu.VMEM((_L,), jnp.float32),
            pltpu.SemaphoreType.DMA,
        ],
    )
    def gather_sum(flat_hbm, c_hbm, out_hbm, c_v, idx_v, val_v, acc_v, sem):
        wid = lax.axis_index("s") * _NC + lax.axis_index("c")
        base = wid * bw
        pltpu.sync_copy(c_hbm.at[pl.ds(base, bw)], c_v)
        lanes = lax.iota(jnp.int32, (_L,))
        for j in range(nchunk):
            for k in range(_IDX_ROW // _L):
                r0 = j * _IDX_ROW + k * _L
                idx_v[j, pl.ds(k * _L, _L)] = (
                    c_v[pl.ds(r0, _L)] + (lanes + (base + r0)) * t
                )
        copies = [
            pltpu.async_copy(flat_hbm.at[idx_v.at[j]], val_v.at[j], sem)
            for j in range(nchunk)
        ]
        for cp in copies:
            cp.wait()
        acc = jnp.zeros((_L,), jnp.float32)
        for j in range(nchunk):
            for k in range(_IDX_ROW // _L):
                acc = acc + val_v[j, pl.ds(k * _L, _L)]
        acc_v[...] = acc
        pltpu.sync_copy(acc_v, out_hbm.at[wid])

    return gather_sum


def kernel(lambda_t, Lambda_t, c):
    b, t = Lambda_t.shape
    s2 = _sq_sum(Lambda_t, _BLOCK_ROWS)[0, 0]
    partials = _make_gather_sum(b, t)(Lambda_t.reshape(-1), c.reshape(-1))
    g = jnp.sum(partials)
    return (s2 - 2.0 * g + jnp.float32(b)) / jnp.float32(b * t)


# speedup vs baseline: 1.1738x; 1.1738x over previous
import functools

import jax
import jax.numpy as jnp
from jax import lax
from jax.experimental import pallas as pl
from jax.experimental.pallas import tpu as pltpu
from jax.experimental.pallas import tpu_sc as plsc

_NC = 2
_NS = 16
_L = 16

_BLOCK_ROWS = 1024
_IDX_ROW = 128


def _sq_sum_body(x_ref, out_ref):
    @pl.when(pl.program_id(0) == 0)
    def _init():
        out_ref[0, 0] = jnp.float32(0.0)

    x = x_ref[...]
    out_ref[0, 0] += jnp.sum(x * x)


@functools.partial(jax.jit, static_argnums=(1,))
def _sq_sum(x, block_rows):
    b, t = x.shape
    return pl.pallas_call(
        _sq_sum_body,
        grid=(b // block_rows,),
        in_specs=[pl.BlockSpec((block_rows, t), lambda i: (i, 0))],
        out_specs=pl.BlockSpec(memory_space=pltpu.SMEM),
        out_shape=jax.ShapeDtypeStruct((1, 1), jnp.float32),
    )(x)


@functools.cache
def _make_gather_sum(b, t):
    nw = _NC * _NS
    bw = b // nw
    nchunk = bw // _IDX_ROW
    mesh = plsc.VectorSubcoreMesh(core_axis_name="c", subcore_axis_name="s")

    @functools.partial(
        pl.kernel,
        mesh=mesh,
        out_type=jax.ShapeDtypeStruct((nw, _L), jnp.float32),
        scratch_types=[
            pltpu.VMEM((bw,), jnp.int32),
            pltpu.VMEM((nchunk, _IDX_ROW), jnp.int32),
            pltpu.VMEM((nchunk, _IDX_ROW), jnp.float32),
            pltpu.VMEM((_L,), jnp.float32),
            pltpu.SemaphoreType.DMA,
        ],
    )
    def gather_sum(flat_hbm, c_hbm, out_hbm, c_v, idx_v, val_v, acc_v, sem):
        wid = lax.axis_index("s") * _NC + lax.axis_index("c")
        base = wid * bw
        pltpu.sync_copy(c_hbm.at[pl.ds(base, bw)], c_v)
        lanes = lax.iota(jnp.int32, _L)
        for j in range(nchunk):
            for k in range(_IDX_ROW // _L):
                r0 = j * _IDX_ROW + k * _L
                idx_v[j, pl.ds(k * _L, _L)] = (
                    c_v[pl.ds(r0, _L)] + (lanes + (base + r0)) * t
                )
        copies = [
            pltpu.async_copy(flat_hbm.at[idx_v.at[j]], val_v.at[j], sem)
            for j in range(nchunk)
        ]
        for cp in copies:
            cp.wait()
        acc = jnp.zeros((_L,), jnp.float32)
        for j in range(nchunk):
            for k in range(_IDX_ROW // _L):
                acc = acc + val_v[j, pl.ds(k * _L, _L)]
        acc_v[...] = acc
        pltpu.sync_copy(acc_v, out_hbm.at[wid])

    return gather_sum


def kernel(lambda_t, Lambda_t, c):
    b, t = Lambda_t.shape
    s2 = _sq_sum(Lambda_t, _BLOCK_ROWS)[0, 0]
    partials = _make_gather_sum(b, t)(Lambda_t.reshape(-1), c.reshape(-1))
    g = jnp.sum(partials)
    return (s2 - 2.0 * g + jnp.float32(b)) / jnp.float32(b * t)
